# Initial kernel scaffold; baseline (speedup 1.0000x reference)
import jax
import jax.numpy as jnp
from jax.experimental import pallas as pl


def kernel(x, edge_index, edge_attr, agent_state, pool_batch, Wl1, Wr1, We1, att1, b1, Wl2, Wr2, We2, att2, b2, Wl3, Wr3, We3, att3, b3, Wg, bg, Wgf1, bgf1, Wgf2, bgf2, Wgf3, bgf3, Wgf4, bgf4, Waf1, baf1, Waf2, baf2, Waf3, baf3, Waf4, baf4, Wo1, bo1, Wo2, bo2, Wo3, bo3):
    raise NotImplementedError("write your pallas kernel here")


# speedup vs baseline: 1.1090x; 1.1090x over previous
import jax
import jax.numpy as jnp
from jax import lax
from jax.experimental import pallas as pl

N, E, G = 10000, 160000, 16
KSHIFT = 50.0
EPS = 1e-30
_f32 = jnp.float32

_NB = 10
_BR = N // _NB


def _proj_kernel(x_ref, wl_ref, wr_ref, xl_ref, xr_ref):
    xb = x_ref[...]
    xl_ref[...] = jnp.dot(xb, wl_ref[...], preferred_element_type=_f32)
    xr_ref[...] = jnp.dot(xb, wr_ref[...], preferred_element_type=_f32)


def _proj_call(x, wl, wr):
    din = x.shape[1]
    return pl.pallas_call(
        _proj_kernel,
        grid=(_NB,),
        in_specs=[
            pl.BlockSpec((_BR, din), lambda i: (i, 0)),
            pl.BlockSpec((din, 256), lambda i: (0, 0)),
            pl.BlockSpec((din, 256), lambda i: (0, 0)),
        ],
        out_specs=[
            pl.BlockSpec((_BR, 256), lambda i: (i, 0)),
            pl.BlockSpec((_BR, 256), lambda i: (i, 0)),
        ],
        out_shape=[
            jax.ShapeDtypeStruct((N, 256), _f32),
            jax.ShapeDtypeStruct((N, 256), _f32),
        ],
    )(x, wl, wr)


def _fin_kernel(msum_ref, den_ref, bias_ref, out_ref):
    out_ref[...] = jnp.maximum(
        msum_ref[...] / (den_ref[...] + EPS) + bias_ref[...], 0.0)


def _fin_call(msum, den_wide, bias):
    return pl.pallas_call(
        _fin_kernel,
        grid=(_NB,),
        in_specs=[
            pl.BlockSpec((_BR, 256), lambda i: (i, 0)),
            pl.BlockSpec((_BR, 256), lambda i: (i, 0)),
            pl.BlockSpec((1, 256), lambda i: (0, 0)),
        ],
        out_specs=pl.BlockSpec((_BR, 256), lambda i: (i, 0)),
        out_shape=jax.ShapeDtypeStruct((N, 256), _f32),
    )(msum, den_wide, bias)


def _head_kernel(h_ref, pb_ref, wg_ref, bg_ref, ag_ref,
                 wgf1, bgf1, wgf2, bgf2, wgf3, bgf3, wgf4, bgf4,
                 waf1, baf1, waf2, baf2, waf3, baf3, waf4, baf4,
                 wo1, bo1, wo2, bo2, wo3, bo3, out_ref):
    h = h_ref[...]
    gate = jnp.dot(h, wg_ref[...], preferred_element_type=_f32)[:, 0] + bg_ref[0]
    pb = pb_ref[...]
    oh = pb[:, None] == lax.broadcasted_iota(jnp.int32, (N, G), 1)
    gfull = jnp.where(oh, gate[:, None], -jnp.inf)
    m = jnp.max(gfull, axis=0)
    m = jnp.where(jnp.isfinite(m), m, 0.0)
    mn = jnp.sum(jnp.where(oh, m[None, :], 0.0), axis=1)
    gexp = jnp.exp(gate - mn)
    deng = jnp.sum(jnp.where(oh, gexp[:, None], 0.0), axis=0)
    dn = jnp.sum(jnp.where(oh, deng[None, :], 0.0), axis=1)
    w = gexp / (dn + 1e-16)
    wh = h * w[:, None]
    pooled = lax.dot_general(oh.astype(_f32), wh,
                             (((0,), (0,)), ((), ())),
                             preferred_element_type=_f32)

    def mlp(v, layers):
        nl = len(layers)
        for i, (wt, bt) in enumerate(layers):
            v = jnp.dot(v, wt[...], preferred_element_type=_f32) + bt[...]
            if i < nl - 1:
                v = jnp.maximum(v, 0.0)
        return v

    gg = mlp(pooled, [(wgf1, bgf1), (wgf2, bgf2), (wgf3, bgf3), (wgf4, bgf4)])
    aa = mlp(ag_ref[...], [(waf1, baf1), (waf2, baf2), (waf3, baf3), (waf4, baf4)])
    fused = jnp.concatenate([gg, aa], axis=-1)
    out_ref[...] = mlp(fused, [(wo1, bo1), (wo2, bo2), (wo3, bo3)])


def _edge_layer(xl, xr, src, dst, eav, We, att, H, C):
    xlh = xl.reshape(N, H, C)
    ea = (eav[:, None] * We[0]).reshape(E, H, C)
    e = xlh[src] + xr.reshape(N, H, C)[dst] + ea
    e = jnp.maximum(e, 0.2 * e)
    logits = (e * att[None, :, :]).sum(-1)
    a = jnp.exp(logits - KSHIFT)
    den = jax.ops.segment_sum(a, dst, num_segments=N)
    msum = jax.ops.segment_sum(a[:, :, None] * xlh[src], dst, num_segments=N)
    den_wide = jnp.broadcast_to(den[:, :, None], (N, H, C)).reshape(N, H * C)
    return msum.reshape(N, H * C), den_wide


def kernel(x, edge_index, edge_attr, agent_state, pool_batch, Wl1, Wr1, We1, att1, b1, Wl2, Wr2, We2, att2, b2, Wl3, Wr3, We3, att3, b3, Wg, bg, Wgf1, bgf1, Wgf2, bgf2, Wgf3, bgf3, Wgf4, bgf4, Waf1, baf1, Waf2, baf2, Waf3, baf3, Waf4, baf4, Wo1, bo1, Wo2, bo2, Wo3, bo3):
    src, dst = edge_index[0], edge_index[1]
    eav = edge_attr[:, 0]

    xl, xr = _proj_call(x, Wl1, Wr1)
    msum, denw = _edge_layer(xl, xr, src, dst, eav, We1, att1, 4, 64)
    h = _fin_call(msum, denw, b1.reshape(1, 256))

    xl, xr = _proj_call(h, Wl2, Wr2)
    msum, denw = _edge_layer(xl, xr, src, dst, eav, We2, att2, 4, 64)
    h = _fin_call(msum, denw, b2.reshape(1, 256))

    xl, xr = _proj_call(h, Wl3, Wr3)
    msum, denw = _edge_layer(xl, xr, src, dst, eav, We3, att3, 2, 128)
    h = _fin_call(msum, denw, b3.reshape(1, 256))

    head_ws = [Wgf1, bgf1, Wgf2, bgf2, Wgf3, bgf3, Wgf4, bgf4,
               Waf1, baf1, Waf2, baf2, Waf3, baf3, Waf4, baf4,
               Wo1, bo1, Wo2, bo2, Wo3, bo3]
    return pl.pallas_call(
        _head_kernel,
        out_shape=jax.ShapeDtypeStruct((G, 8), _f32),
    )(h, pool_batch, Wg, bg, agent_state, *head_ws)
